# baseline (speedup 1.0000x reference)
import functools

import jax
import jax.numpy as jnp
from jax import lax
from jax.experimental import pallas as pl
from jax.experimental.pallas import tpu as pltpu
from jax.experimental.pallas import tpu_sc as plsc

_NC = 2
_NS = 16
_NW = _NC * _NS

_LANES = 16


def _make_pool(B, S, E, V):
    b_per_w = B // _NW
    chunks = (128, S - 128) if S > 128 else (S,)
    offs = (0, 128)
    groups = E // _LANES

    mesh = plsc.VectorSubcoreMesh(core_axis_name="c", subcore_axis_name="s")

    @functools.partial(
        pl.kernel,
        mesh=mesh,
        out_type=jax.ShapeDtypeStruct((B * E,), jnp.float32),
        scratch_types=[
            pltpu.VMEM((b_per_w * S,), jnp.int32),
            pltpu.VMEM((4 * S, E), jnp.float32),
            pltpu.VMEM((b_per_w * E,), jnp.float32),
            pltpu.SemaphoreType.DMA((4,)),
        ],
        compiler_params=pltpu.CompilerParams(use_tc_tiling_on_sc=False),
    )
    def pool(x_hbm, table_hbm, out_hbm, idx_v, banks_v, acc_v, sems):
        wid = lax.axis_index("s") * _NC + lax.axis_index("c")
        base = wid * b_per_w
        pltpu.sync_copy(
            x_hbm.at[pl.ds(pl.multiple_of(base * S, 8), b_per_w * S)], idx_v
        )

        def src(i, c):
            off = pl.multiple_of(i * S + offs[c], 8)
            return table_hbm.at[idx_v.at[pl.ds(off, chunks[c])]]

        def bank_dst(i, c):
            rowbase = pl.multiple_of((i & 3) * S, 8)
            return banks_v.at[pl.ds(rowbase + offs[c], chunks[c]), :]

        def fire(i):
            for c in range(len(chunks)):
                pltpu.async_copy(src(i, c), bank_dst(i, c), sems.at[i & 3])

        def reduce_store(i):
            for c in range(len(chunks)):
                pltpu.make_async_copy(
                    table_hbm.at[pl.ds(0, chunks[c])],
                    bank_dst(i, c),
                    sems.at[i & 3],
                ).wait()
            rowbase = pl.multiple_of((i & 3) * S, 8)
            accs = [jnp.zeros((_LANES,), jnp.float32) for _ in range(groups)]
            for j in range(S):
                for g in range(groups):
                    accs[g] = (
                        accs[g] + banks_v[rowbase + j, pl.ds(g * _LANES, _LANES)]
                    )
            scale = jnp.float32(1.0 / S)
            for g in range(groups):
                aoff = pl.multiple_of(i * E + g * _LANES, 8)
                acc_v[pl.ds(aoff, _LANES)] = accs[g] * scale

        fire(0)
        fire(1)
        fire(2)

        def body(i, carry):
            @pl.when(i + 3 < b_per_w)
            def _():
                fire(i + 3)

            reduce_store(i)
            return carry

        lax.fori_loop(0, b_per_w, body, 0)
        pltpu.sync_copy(
            acc_v, out_hbm.at[pl.ds(pl.multiple_of(base * E, 8), b_per_w * E)]
        )

    return pool


def _mm_body(p_ref, w_ref, b_ref, o_ref):
    o_ref[...] = (
        lax.dot_general(
            p_ref[...],
            w_ref[...],
            (((1,), (1,)), ((), ())),
            preferred_element_type=jnp.float32,
        )
        + b_ref[...]
    )


def _matmul(pooled, lin_w, lin_b2d, v_blk=2048):
    B, E = pooled.shape
    V = lin_w.shape[0]
    nb = pl.cdiv(V, v_blk)
    return pl.pallas_call(
        _mm_body,
        grid=(nb,),
        in_specs=[
            pl.BlockSpec((B, E), lambda i: (0, 0)),
            pl.BlockSpec((v_blk, E), lambda i: (i, 0)),
            pl.BlockSpec((1, v_blk), lambda i: (0, i)),
        ],
        out_specs=pl.BlockSpec((B, v_blk), lambda i: (0, i)),
        out_shape=jax.ShapeDtypeStruct((B, V), jnp.float32),
        compiler_params=pltpu.CompilerParams(
            dimension_semantics=("parallel",),
        ),
    )(pooled, lin_w, lin_b2d)


@jax.jit
def kernel(x, emb_table, lin_w, lin_b):
    B, S = x.shape
    V, E = emb_table.shape
    H = 2
    Bh = B // H
    pool = _make_pool(Bh, S, E, V)
    lin_b2 = lin_b.reshape(1, V)
    halves = []
    for h in range(H):
        xh = lax.slice(x, (h * Bh, 0), ((h + 1) * Bh, S)).reshape(Bh * S)
        pooled = pool(xh, emb_table).reshape(Bh, E)
        halves.append(_matmul(pooled, lin_w, lin_b2))
    return jnp.concatenate(halves, axis=0)


# speedup vs baseline: 1.2638x; 1.2638x over previous
import functools

import jax
import jax.numpy as jnp
from jax import lax
from jax.experimental import pallas as pl
from jax.experimental.pallas import tpu as pltpu
from jax.experimental.pallas import tpu_sc as plsc

_NC = 2
_NS = 16
_NW = _NC * _NS

_LANES = 16


def _make_pool(B, S, E, V):
    b_per_w = B // _NW
    chunks = (128, S - 128) if S > 128 else (S,)
    offs = (0, 128)
    groups = E // _LANES

    mesh = plsc.VectorSubcoreMesh(core_axis_name="c", subcore_axis_name="s")

    @functools.partial(
        pl.kernel,
        mesh=mesh,
        out_type=jax.ShapeDtypeStruct((B * E,), jnp.float32),
        scratch_types=[
            pltpu.VMEM((b_per_w * S,), jnp.int32),
            pltpu.VMEM((4 * S, E), jnp.float32),
            pltpu.VMEM((b_per_w * E,), jnp.float32),
            pltpu.SemaphoreType.DMA((4,)),
        ],
        compiler_params=pltpu.CompilerParams(use_tc_tiling_on_sc=False),
    )
    def pool(x_hbm, table_hbm, out_hbm, idx_v, banks_v, acc_v, sems):
        wid = lax.axis_index("s") * _NC + lax.axis_index("c")
        base = wid * b_per_w
        pltpu.sync_copy(
            x_hbm.at[pl.ds(pl.multiple_of(base * S, 8), b_per_w * S)], idx_v
        )

        def src(i, c):
            off = pl.multiple_of(i * S + offs[c], 8)
            return table_hbm.at[idx_v.at[pl.ds(off, chunks[c])]]

        def bank_dst(i, c):
            rowbase = pl.multiple_of((i & 3) * S, 8)
            return banks_v.at[pl.ds(rowbase + offs[c], chunks[c]), :]

        def fire(i):
            for c in range(len(chunks)):
                pltpu.async_copy(src(i, c), bank_dst(i, c), sems.at[i & 3])

        def reduce_store(i):
            for c in range(len(chunks)):
                pltpu.make_async_copy(
                    table_hbm.at[pl.ds(0, chunks[c])],
                    bank_dst(i, c),
                    sems.at[i & 3],
                ).wait()
            rowbase = pl.multiple_of((i & 3) * S, 8)
            accs = [jnp.zeros((_LANES,), jnp.float32) for _ in range(groups)]
            for j in range(S):
                for g in range(groups):
                    accs[g] = (
                        accs[g] + banks_v[rowbase + j, pl.ds(g * _LANES, _LANES)]
                    )
            scale = jnp.float32(1.0 / S)
            for g in range(groups):
                aoff = pl.multiple_of(i * E + g * _LANES, 8)
                acc_v[pl.ds(aoff, _LANES)] = accs[g] * scale

        fire(0)
        fire(1)
        fire(2)

        def body(i, carry):
            @pl.when(i + 3 < b_per_w)
            def _():
                fire(i + 3)

            reduce_store(i)
            return carry

        lax.fori_loop(0, b_per_w, body, 0)
        pltpu.sync_copy(
            acc_v, out_hbm.at[pl.ds(pl.multiple_of(base * E, 8), b_per_w * E)]
        )

    return pool


def _mm_body(p_ref, w_ref, b_ref, o_ref):
    o_ref[...] = (
        lax.dot_general(
            p_ref[...],
            w_ref[...],
            (((1,), (1,)), ((), ())),
            preferred_element_type=jnp.float32,
        )
        + b_ref[...]
    )


def _mm_body_carry(c_ref, p_ref, w_ref, b_ref, o_ref):
    del c_ref
    _mm_body(p_ref, w_ref, b_ref, o_ref)


def _matmul_rows(pooled, lin_w, lin_b2d, B, h, carry=None, v_blk=2048):
    Bh, E = pooled.shape
    V = lin_w.shape[0]
    nb = pl.cdiv(V, v_blk)
    in_specs = [
        pl.BlockSpec((Bh, E), lambda i: (0, 0)),
        pl.BlockSpec((v_blk, E), lambda i: (i, 0)),
        pl.BlockSpec((1, v_blk), lambda i: (0, i)),
    ]
    args = [pooled, lin_w, lin_b2d]
    body = _mm_body
    aliases = {}
    if carry is not None:
        in_specs = [pl.BlockSpec(memory_space=pl.ANY)] + in_specs
        args = [carry] + args
        body = _mm_body_carry
        aliases = {0: 0}
    return pl.pallas_call(
        body,
        grid=(nb,),
        in_specs=in_specs,
        out_specs=pl.BlockSpec((Bh, v_blk), lambda i: (h, i)),
        out_shape=jax.ShapeDtypeStruct((B, V), jnp.float32),
        input_output_aliases=aliases,
        compiler_params=pltpu.CompilerParams(
            dimension_semantics=("arbitrary",),
        ),
    )(*args)


@jax.jit
def kernel(x, emb_table, lin_w, lin_b):
    B, S = x.shape
    V, E = emb_table.shape
    H = 2
    Bh = B // H
    pool = _make_pool(Bh, S, E, V)
    lin_b2 = lin_b.reshape(1, V)
    pooleds = []
    for h in range(H):
        xh = lax.slice(x, (h * Bh, 0), ((h + 1) * Bh, S)).reshape(Bh * S)
        pooleds.append(pool(xh, emb_table).reshape(Bh, E))
    out = _matmul_rows(pooleds[0], lin_w, lin_b2, B, 0)
    for h in range(1, H):
        out = _matmul_rows(pooleds[h], lin_w, lin_b2, B, h, carry=out)
    return out


# speedup vs baseline: 2.2970x; 1.8175x over previous
import functools

import jax
import jax.numpy as jnp
from jax import lax
from jax.experimental import pallas as pl
from jax.experimental.pallas import tpu as pltpu
from jax.experimental.pallas import tpu_sc as plsc

_NC = 2
_NS = 16
_NW = _NC * _NS

_LANES = 16


def _make_pool(B, S, E, V):
    b_per_w = B // _NW
    chunks = (128, S - 128) if S > 128 else (S,)
    offs = (0, 128)
    groups = E // _LANES

    mesh = plsc.VectorSubcoreMesh(core_axis_name="c", subcore_axis_name="s")

    @functools.partial(
        pl.kernel,
        mesh=mesh,
        out_type=jax.ShapeDtypeStruct((B * E,), jnp.float32),
        scratch_types=[
            pltpu.VMEM((b_per_w * S,), jnp.int32),
            pltpu.VMEM((4 * S, E), jnp.float32),
            pltpu.VMEM((b_per_w * E,), jnp.float32),
            pltpu.SemaphoreType.DMA((4,)),
        ],
        compiler_params=pltpu.CompilerParams(use_tc_tiling_on_sc=False),
    )
    def pool(x_hbm, table_hbm, out_hbm, idx_v, banks_v, acc_v, sems):
        wid = lax.axis_index("s") * _NC + lax.axis_index("c")
        base = wid * b_per_w
        pltpu.sync_copy(
            x_hbm.at[pl.ds(pl.multiple_of(base * S, 8), b_per_w * S)], idx_v
        )

        def src(i, c):
            off = pl.multiple_of(i * S + offs[c], 8)
            return table_hbm.at[idx_v.at[pl.ds(off, chunks[c])]]

        def bank_dst(i, c):
            rowbase = pl.multiple_of((i & 3) * S, 8)
            return banks_v.at[pl.ds(rowbase + offs[c], chunks[c]), :]

        def fire(i):
            for c in range(len(chunks)):
                pltpu.async_copy(src(i, c), bank_dst(i, c), sems.at[i & 3])

        def reduce_store(i):
            for c in range(len(chunks)):
                pltpu.make_async_copy(
                    table_hbm.at[pl.ds(0, chunks[c])],
                    bank_dst(i, c),
                    sems.at[i & 3],
                ).wait()
            rowbase = pl.multiple_of((i & 3) * S, 8)
            accs = [jnp.zeros((_LANES,), jnp.float32) for _ in range(groups)]
            for j in range(S):
                for g in range(groups):
                    accs[g] = (
                        accs[g] + banks_v[rowbase + j, pl.ds(g * _LANES, _LANES)]
                    )
            scale = jnp.float32(1.0 / S)
            for g in range(groups):
                aoff = pl.multiple_of(i * E + g * _LANES, 8)
                acc_v[pl.ds(aoff, _LANES)] = accs[g] * scale

        fire(0)
        fire(1)
        fire(2)

        def body(i, carry):
            @pl.when(i + 3 < b_per_w)
            def _():
                fire(i + 3)

            reduce_store(i)
            return carry

        lax.fori_loop(0, b_per_w, body, 0)
        pltpu.sync_copy(
            acc_v, out_hbm.at[pl.ds(pl.multiple_of(base * E, 8), b_per_w * E)]
        )

    return pool


def _mm_t_body(w_ref, p_ref, b_ref, o_ref):
    o_ref[...] = (
        lax.dot_general(
            w_ref[...],
            p_ref[...],
            (((1,), (1,)), ((), ())),
            preferred_element_type=jnp.float32,
        )
        + b_ref[...]
    )


def _mm_t_body_carry(c_ref, w_ref, p_ref, b_ref, o_ref):
    del c_ref
    _mm_t_body(w_ref, p_ref, b_ref, o_ref)


def _matmul_cols(pooled, lin_w, lin_bc, B, h, carry=None, v_blk=2048):
    Bh, E = pooled.shape
    V = lin_w.shape[0]
    nb = pl.cdiv(V, v_blk)
    in_specs = [
        pl.BlockSpec((v_blk, E), lambda i: (i, 0)),
        pl.BlockSpec((Bh, E), lambda i: (0, 0)),
        pl.BlockSpec((v_blk, 1), lambda i: (i, 0)),
    ]
    args = [lin_w, pooled, lin_bc]
    body = _mm_t_body
    aliases = {}
    if carry is not None:
        in_specs = [pl.BlockSpec(memory_space=pl.ANY)] + in_specs
        args = [carry] + args
        body = _mm_t_body_carry
        aliases = {0: 0}
    return pl.pallas_call(
        body,
        grid=(nb,),
        in_specs=in_specs,
        out_specs=pl.BlockSpec((v_blk, Bh), lambda i: (i, h)),
        out_shape=jax.ShapeDtypeStruct((V, B), jnp.float32),
        input_output_aliases=aliases,
        compiler_params=pltpu.CompilerParams(
            dimension_semantics=("arbitrary",),
        ),
    )(*args)


@jax.jit
def kernel(x, emb_table, lin_w, lin_b):
    B, S = x.shape
    V, E = emb_table.shape
    H = 2
    Bh = B // H
    pool = _make_pool(Bh, S, E, V)
    lin_bc = lin_b.reshape(V, 1)
    pooleds = []
    for h in range(H):
        xh = lax.slice(x, (h * Bh, 0), ((h + 1) * Bh, S)).reshape(Bh * S)
        pooleds.append(pool(xh, emb_table).reshape(Bh, E))
    out_t = _matmul_cols(pooleds[0], lin_w, lin_bc, B, 0)
    for h in range(1, H):
        out_t = _matmul_cols(pooleds[h], lin_w, lin_bc, B, h, carry=out_t)
    return out_t.T


# speedup vs baseline: 2.6992x; 1.1751x over previous
import functools

import jax
import jax.numpy as jnp
from jax import lax
from jax.experimental import pallas as pl
from jax.experimental.pallas import tpu as pltpu
from jax.experimental.pallas import tpu_sc as plsc

_NC = 2
_NS = 16
_NW = _NC * _NS

_LANES = 16


def _make_pool(B, S, E, V):
    b_per_w = B // _NW
    chunks = (128, S - 128) if S > 128 else (S,)
    offs = (0, 128)
    groups = E // _LANES

    mesh = plsc.VectorSubcoreMesh(core_axis_name="c", subcore_axis_name="s")

    @functools.partial(
        pl.kernel,
        mesh=mesh,
        out_type=jax.ShapeDtypeStruct((B * E,), jnp.float32),
        scratch_types=[
            pltpu.VMEM((b_per_w * S,), jnp.int32),
            pltpu.VMEM((4 * S, E), jnp.float32),
            pltpu.VMEM((b_per_w * E,), jnp.float32),
            pltpu.SemaphoreType.DMA((4,)),
        ],
        compiler_params=pltpu.CompilerParams(use_tc_tiling_on_sc=False),
    )
    def pool(x_hbm, table_hbm, out_hbm, idx_v, banks_v, acc_v, sems):
        wid = lax.axis_index("s") * _NC + lax.axis_index("c")
        base = wid * b_per_w
        pltpu.sync_copy(
            x_hbm.at[pl.ds(pl.multiple_of(base * S, 8), b_per_w * S)], idx_v
        )

        def src(i, c):
            off = pl.multiple_of(i * S + offs[c], 8)
            return table_hbm.at[idx_v.at[pl.ds(off, chunks[c])]]

        def bank_dst(i, c):
            rowbase = pl.multiple_of((i & 3) * S, 8)
            return banks_v.at[pl.ds(rowbase + offs[c], chunks[c]), :]

        def fire(i):
            for c in range(len(chunks)):
                pltpu.async_copy(src(i, c), bank_dst(i, c), sems.at[i & 3])

        def reduce_store(i):
            for c in range(len(chunks)):
                pltpu.make_async_copy(
                    table_hbm.at[pl.ds(0, chunks[c])],
                    bank_dst(i, c),
                    sems.at[i & 3],
                ).wait()
            rowbase = pl.multiple_of((i & 3) * S, 8)
            accs = [jnp.zeros((_LANES,), jnp.float32) for _ in range(groups)]
            for j in range(S):
                for g in range(groups):
                    accs[g] = (
                        accs[g] + banks_v[rowbase + j, pl.ds(g * _LANES, _LANES)]
                    )
            scale = jnp.float32(1.0 / S)
            for g in range(groups):
                aoff = pl.multiple_of(i * E + g * _LANES, 8)
                acc_v[pl.ds(aoff, _LANES)] = accs[g] * scale

        fire(0)
        fire(1)
        fire(2)

        def body(i, carry):
            @pl.when(i + 3 < b_per_w)
            def _():
                fire(i + 3)

            reduce_store(i)
            return carry

        lax.fori_loop(0, b_per_w, body, 0)
        pltpu.sync_copy(
            acc_v, out_hbm.at[pl.ds(pl.multiple_of(base * E, 8), b_per_w * E)]
        )

    return pool


def _mm_t_body(w_ref, p_ref, b_ref, o_ref):
    o_ref[...] = (
        lax.dot_general(
            w_ref[...],
            p_ref[...],
            (((1,), (1,)), ((), ())),
            preferred_element_type=jnp.float32,
        )
        + b_ref[...]
    )


def _mm_t_body_carry(c_ref, w_ref, p_ref, b_ref, o_ref):
    del c_ref
    _mm_t_body(w_ref, p_ref, b_ref, o_ref)


def _matmul_cols(pooled, lin_w, lin_bc, B, h, carry=None, v_blk=2048):
    Bh, E = pooled.shape
    V = lin_w.shape[0]
    nb = pl.cdiv(V, v_blk)
    in_specs = [
        pl.BlockSpec((v_blk, E), lambda i: (i, 0)),
        pl.BlockSpec((Bh, E), lambda i: (0, 0)),
        pl.BlockSpec((v_blk, 1), lambda i: (i, 0)),
    ]
    args = [lin_w, pooled, lin_bc]
    body = _mm_t_body
    aliases = {}
    if carry is not None:
        in_specs = [pl.BlockSpec(memory_space=pl.ANY)] + in_specs
        args = [carry] + args
        body = _mm_t_body_carry
        aliases = {0: 0}
    return pl.pallas_call(
        body,
        grid=(nb,),
        in_specs=in_specs,
        out_specs=pl.BlockSpec((v_blk, Bh), lambda i: (i, h)),
        out_shape=jax.ShapeDtypeStruct((V, B), jnp.float32),
        input_output_aliases=aliases,
        compiler_params=pltpu.CompilerParams(
            dimension_semantics=("arbitrary",),
        ),
    )(*args)


@jax.jit
def kernel(x, emb_table, lin_w, lin_b):
    B, S = x.shape
    V, E = emb_table.shape
    pooled = _make_pool(B, S, E, V)(x.reshape(B * S), emb_table)
    pooled = pooled.reshape(B, E)
    lin_bc = lin_b.reshape(V, 1)
    return _matmul_cols(pooled, lin_w, lin_bc, B, 0).T


# speedup vs baseline: 2.7397x; 1.0150x over previous
import functools

import jax
import jax.numpy as jnp
from jax import lax
from jax.experimental import pallas as pl
from jax.experimental.pallas import tpu as pltpu
from jax.experimental.pallas import tpu_sc as plsc

_NC = 2
_NS = 16
_NW = _NC * _NS

_LANES = 16


def _make_pool(B, S, E, V):
    b_per_w = B // _NW
    chunks = (128, S - 128) if S > 128 else (S,)
    offs = (0, 128)
    groups = E // _LANES

    mesh = plsc.VectorSubcoreMesh(core_axis_name="c", subcore_axis_name="s")

    @functools.partial(
        pl.kernel,
        mesh=mesh,
        out_type=jax.ShapeDtypeStruct((B * E,), jnp.float32),
        scratch_types=[
            pltpu.VMEM((b_per_w * S,), jnp.int32),
            pltpu.VMEM((4 * S, E), jnp.float32),
            pltpu.VMEM((b_per_w * E,), jnp.float32),
            pltpu.SemaphoreType.DMA((4,)),
        ],
        compiler_params=pltpu.CompilerParams(use_tc_tiling_on_sc=False),
    )
    def pool(x_hbm, table_hbm, out_hbm, idx_v, banks_v, acc_v, sems):
        wid = lax.axis_index("s") * _NC + lax.axis_index("c")
        base = wid * b_per_w
        pltpu.sync_copy(
            x_hbm.at[pl.ds(pl.multiple_of(base * S, 8), b_per_w * S)], idx_v
        )

        def src(i, c):
            off = pl.multiple_of(i * S + offs[c], 8)
            return table_hbm.at[idx_v.at[pl.ds(off, chunks[c])]]

        def bank_dst(i, c):
            rowbase = pl.multiple_of((i & 3) * S, 8)
            return banks_v.at[pl.ds(rowbase + offs[c], chunks[c]), :]

        def fire(i):
            for c in range(len(chunks)):
                pltpu.async_copy(src(i, c), bank_dst(i, c), sems.at[i & 3])

        def reduce_store(i):
            for c in range(len(chunks)):
                pltpu.make_async_copy(
                    table_hbm.at[pl.ds(0, chunks[c])],
                    bank_dst(i, c),
                    sems.at[i & 3],
                ).wait()
            rowbase = pl.multiple_of((i & 3) * S, 8)
            accs = [jnp.zeros((_LANES,), jnp.float32) for _ in range(groups)]
            for j in range(S):
                for g in range(groups):
                    accs[g] = (
                        accs[g] + banks_v[rowbase + j, pl.ds(g * _LANES, _LANES)]
                    )
            scale = jnp.float32(1.0 / S)
            for g in range(groups):
                aoff = pl.multiple_of(i * E + g * _LANES, 8)
                acc_v[pl.ds(aoff, _LANES)] = accs[g] * scale

        fire(0)
        fire(1)
        fire(2)

        def body(i, carry):
            @pl.when(i + 3 < b_per_w)
            def _():
                fire(i + 3)

            reduce_store(i)
            return carry

        lax.fori_loop(0, b_per_w, body, 0)
        pltpu.sync_copy(
            acc_v, out_hbm.at[pl.ds(pl.multiple_of(base * E, 8), b_per_w * E)]
        )

    return pool


def _mm_t_body(w_ref, p_ref, b_ref, o_ref):
    o_ref[...] = (
        lax.dot_general(
            w_ref[...],
            p_ref[...],
            (((1,), (1,)), ((), ())),
            preferred_element_type=jnp.float32,
        )
        + b_ref[...]
    )


def _mm_t_body_carry(c_ref, w_ref, p_ref, b_ref, o_ref):
    del c_ref
    _mm_t_body(w_ref, p_ref, b_ref, o_ref)


def _matmul_cols(pooled, lin_w, lin_bc, B, h, carry=None, v_blk=4096):
    Bh, E = pooled.shape
    V = lin_w.shape[0]
    nb = pl.cdiv(V, v_blk)
    in_specs = [
        pl.BlockSpec((v_blk, E), lambda i: (i, 0)),
        pl.BlockSpec((Bh, E), lambda i: (0, 0)),
        pl.BlockSpec((v_blk, 1), lambda i: (i, 0)),
    ]
    args = [lin_w, pooled, lin_bc]
    body = _mm_t_body
    aliases = {}
    if carry is not None:
        in_specs = [pl.BlockSpec(memory_space=pl.ANY)] + in_specs
        args = [carry] + args
        body = _mm_t_body_carry
        aliases = {0: 0}
    return pl.pallas_call(
        body,
        grid=(nb,),
        in_specs=in_specs,
        out_specs=pl.BlockSpec((v_blk, Bh), lambda i: (i, h)),
        out_shape=jax.ShapeDtypeStruct((V, B), jnp.float32),
        input_output_aliases=aliases,
        compiler_params=pltpu.CompilerParams(
            dimension_semantics=("arbitrary",),
        ),
    )(*args)


@jax.jit
def kernel(x, emb_table, lin_w, lin_b):
    B, S = x.shape
    V, E = emb_table.shape
    pooled = _make_pool(B, S, E, V)(x.reshape(B * S), emb_table)
    pooled = pooled.reshape(B, E)
    lin_bc = lin_b.reshape(V, 1)
    return _matmul_cols(pooled, lin_w, lin_bc, B, 0).T


# speedup vs baseline: 2.7437x; 1.0014x over previous
import functools

import jax
import jax.numpy as jnp
from jax import lax
from jax.experimental import pallas as pl
from jax.experimental.pallas import tpu as pltpu
from jax.experimental.pallas import tpu_sc as plsc

_NC = 2
_NS = 16
_NW = _NC * _NS

_LANES = 16


def _make_pool(B, S, E, V):
    b_per_w = B // _NW
    chunks = (128, S - 128) if S > 128 else (S,)
    offs = (0, 128)
    groups = E // _LANES

    mesh = plsc.VectorSubcoreMesh(core_axis_name="c", subcore_axis_name="s")

    @functools.partial(
        pl.kernel,
        mesh=mesh,
        out_type=jax.ShapeDtypeStruct((B * E,), jnp.float32),
        scratch_types=[
            pltpu.VMEM((b_per_w * S,), jnp.int32),
            pltpu.VMEM((8 * S, E), jnp.float32),
            pltpu.VMEM((b_per_w * E,), jnp.float32),
            pltpu.SemaphoreType.DMA((8,)),
        ],
        compiler_params=pltpu.CompilerParams(use_tc_tiling_on_sc=False),
    )
    def pool(x_hbm, table_hbm, out_hbm, idx_v, banks_v, acc_v, sems):
        wid = lax.axis_index("s") * _NC + lax.axis_index("c")
        base = wid * b_per_w
        pltpu.sync_copy(
            x_hbm.at[pl.ds(pl.multiple_of(base * S, 8), b_per_w * S)], idx_v
        )

        def src(i, c):
            off = pl.multiple_of(i * S + offs[c], 8)
            return table_hbm.at[idx_v.at[pl.ds(off, chunks[c])]]

        def bank_dst(i, c):
            rowbase = pl.multiple_of((i & 7) * S, 8)
            return banks_v.at[pl.ds(rowbase + offs[c], chunks[c]), :]

        def fire(i):
            for c in range(len(chunks)):
                pltpu.async_copy(src(i, c), bank_dst(i, c), sems.at[i & 7])

        def reduce_store(i):
            for c in range(len(chunks)):
                pltpu.make_async_copy(
                    table_hbm.at[pl.ds(0, chunks[c])],
                    bank_dst(i, c),
                    sems.at[i & 7],
                ).wait()
            rowbase = pl.multiple_of((i & 7) * S, 8)
            accs = [jnp.zeros((_LANES,), jnp.float32) for _ in range(groups)]
            for j in range(S):
                for g in range(groups):
                    accs[g] = (
                        accs[g] + banks_v[rowbase + j, pl.ds(g * _LANES, _LANES)]
                    )
            scale = jnp.float32(1.0 / S)
            for g in range(groups):
                aoff = pl.multiple_of(i * E + g * _LANES, 8)
                acc_v[pl.ds(aoff, _LANES)] = accs[g] * scale

        for k in range(7):
            fire(k)

        def body(i, carry):
            @pl.when(i + 7 < b_per_w)
            def _():
                fire(i + 7)

            reduce_store(i)
            return carry

        lax.fori_loop(0, b_per_w, body, 0)
        pltpu.sync_copy(
            acc_v, out_hbm.at[pl.ds(pl.multiple_of(base * E, 8), b_per_w * E)]
        )

    return pool


def _mm_t_body(w_ref, p_ref, b_ref, o_ref):
    o_ref[...] = (
        lax.dot_general(
            w_ref[...],
            p_ref[...],
            (((1,), (1,)), ((), ())),
            preferred_element_type=jnp.float32,
        )
        + b_ref[...]
    )


def _mm_t_body_carry(c_ref, w_ref, p_ref, b_ref, o_ref):
    del c_ref
    _mm_t_body(w_ref, p_ref, b_ref, o_ref)


def _matmul_cols(pooled, lin_w, lin_bc, B, h, carry=None, v_blk=4096):
    Bh, E = pooled.shape
    V = lin_w.shape[0]
    nb = pl.cdiv(V, v_blk)
    in_specs = [
        pl.BlockSpec((v_blk, E), lambda i: (i, 0)),
        pl.BlockSpec((Bh, E), lambda i: (0, 0)),
        pl.BlockSpec((v_blk, 1), lambda i: (i, 0)),
    ]
    args = [lin_w, pooled, lin_bc]
    body = _mm_t_body
    aliases = {}
    if carry is not None:
        in_specs = [pl.BlockSpec(memory_space=pl.ANY)] + in_specs
        args = [carry] + args
        body = _mm_t_body_carry
        aliases = {0: 0}
    return pl.pallas_call(
        body,
        grid=(nb,),
        in_specs=in_specs,
        out_specs=pl.BlockSpec((v_blk, Bh), lambda i: (i, h)),
        out_shape=jax.ShapeDtypeStruct((V, B), jnp.float32),
        input_output_aliases=aliases,
        compiler_params=pltpu.CompilerParams(
            dimension_semantics=("arbitrary",),
        ),
    )(*args)


@jax.jit
def kernel(x, emb_table, lin_w, lin_b):
    B, S = x.shape
    V, E = emb_table.shape
    pooled = _make_pool(B, S, E, V)(x.reshape(B * S), emb_table)
    pooled = pooled.reshape(B, E)
    lin_bc = lin_b.reshape(V, 1)
    return _matmul_cols(pooled, lin_w, lin_bc, B, 0).T
